# baseline (speedup 1.0000x reference)
import functools

import jax
import jax.numpy as jnp
import numpy as np
from jax.experimental import pallas as pl

N = 10000
E = 320000
D = 128
DK = 32
NB = 10
MAXR = 3.0

BE = 2560


def _edge_block_body(xs_ref, q_ref, r2_ref, wk1_ref, wk2_ref, wv1_ref,
                     wv2_ref, wklin_ref, wvlin_ref,
                     v_ref, logits_ref, cut_ref):
    r2 = r2_ref[...]
    r = jnp.sqrt(r2 + 1e-9)
    width = MAXR / NB
    centers = jax.lax.broadcasted_iota(jnp.int32, (1, NB), 1).astype(
        jnp.float32) * (MAXR / (NB - 1))
    rbf = jnp.exp(-(((r - centers) / width) ** 2)) * np.sqrt(NB)
    cut_ref[...] = 0.5 * (
        jnp.cos(jnp.pi * jnp.clip(r / MAXR, 0.0, 1.0)) + 1.0)

    a = jax.nn.silu(jnp.dot(rbf, wk1_ref[...],
                            preferred_element_type=jnp.float32))
    wk = jnp.dot(a, wk2_ref[...], preferred_element_type=jnp.float32)
    b = jax.nn.silu(jnp.dot(rbf, wv1_ref[...],
                            preferred_element_type=jnp.float32))
    wv = jnp.dot(b, wv2_ref[...], preferred_element_type=jnp.float32)

    xs = xs_ref[...]
    k = jnp.dot(xs * wk, wklin_ref[...],
                preferred_element_type=jnp.float32)
    logits_ref[...] = jnp.sum(q_ref[...] * k, axis=1,
                              keepdims=True) * (1.0 / np.sqrt(DK))
    v_ref[...] = jnp.dot(xs * wv, wvlin_ref[...],
                         preferred_element_type=jnp.float32)


def _edge_stage(xs, q, r2, Wk1, Wk2, Wv1, Wv2, Wk_lin, Wv_lin):
    nblk = E // BE
    grid = (nblk,)
    edge_spec = lambda cols: pl.BlockSpec((BE, cols), lambda i: (i, 0))
    full = lambda a: pl.BlockSpec(a.shape, lambda i: (0, 0))
    return pl.pallas_call(
        _edge_block_body,
        grid=grid,
        in_specs=[
            edge_spec(D), edge_spec(DK), edge_spec(1),
            full(Wk1), full(Wk2), full(Wv1), full(Wv2),
            full(Wk_lin), full(Wv_lin),
        ],
        out_specs=[edge_spec(D), edge_spec(1), edge_spec(1)],
        out_shape=[
            jax.ShapeDtypeStruct((E, D), jnp.float32),
            jax.ShapeDtypeStruct((E, 1), jnp.float32),
            jax.ShapeDtypeStruct((E, 1), jnp.float32),
        ],
    )(xs, q, r2, Wk1, Wk2, Wv1, Wv2, Wk_lin, Wv_lin)


def kernel(x, pos, edge_index, Wq, Wk_lin, Wv_lin, Wk1, Wk2, Wv1, Wv2):
    src = edge_index[0]
    dst = edge_index[1]
    evec = jnp.take(pos, dst, axis=0) - jnp.take(pos, src, axis=0)
    r2 = jnp.sum(evec * evec, axis=-1, keepdims=True)
    xs = jnp.take(x, src, axis=0)
    q = jnp.take(x @ Wq, dst, axis=0)

    v, logits2, cut2 = _edge_stage(xs, q, r2, Wk1, Wk2, Wv1, Wv2,
                                   Wk_lin, Wv_lin)
    logits = logits2[:, 0]
    cut = cut2[:, 0]

    m = jax.ops.segment_max(logits, dst, num_segments=N)
    m = jnp.where(jnp.isfinite(m), m, 0.0)
    e = cut * jnp.exp(logits - jnp.take(m, dst))
    z = jax.ops.segment_sum(e, dst, num_segments=N)
    alpha = e / (jnp.take(z, dst) + 1e-9)
    w = jnp.sqrt(jax.nn.relu(alpha) + 1e-12)
    out = jax.ops.segment_sum(w[:, None] * v, dst, num_segments=N)
    return out


# speedup vs baseline: 4.6309x; 4.6309x over previous
import jax
import jax.numpy as jnp
import numpy as np
from jax import lax
from jax.experimental import pallas as pl
from jax.experimental.pallas import tpu as pltpu
from jax.experimental.pallas import tpu_sc as plsc

N = 10000
E = 320000
D = 128
DK = 32
NB = 10
MAXR = 3.0

NC = 2
NS = 16
NW = NC * NS
L = 16

NP = 10240
EPW = E // NW
GC = 400
CB = 2000
SL = NP // NS
RT = N // NS
BE = 2560


def _sc_mesh():
    return plsc.VectorSubcoreMesh(
        core_axis_name="c", subcore_axis_name="s",
        num_cores=NC, num_subcores=NS)


_SC_PARAMS = pltpu.CompilerParams(use_tc_tiling_on_sc=False,
                                  needs_layout_passes=False)


def _wid():
    return lax.axis_index("s") * NC + lax.axis_index("c")


def _proj_body(x_ref, wq_ref, q_ref):
    q_ref[...] = jnp.dot(x_ref[...], wq_ref[...],
                         preferred_element_type=jnp.float32)


def _proj_stage(x, Wq):
    return pl.pallas_call(
        _proj_body,
        out_shape=jax.ShapeDtypeStruct((N, DK), jnp.float32),
    )(x, Wq)


DXA = D + 16
DQA = DK + 16


def _rbf_cut(r2):
    r = jnp.sqrt(r2 + 1e-9)
    width = MAXR / NB
    centers = lax.broadcasted_iota(jnp.int32, (1, NB), 1).astype(
        jnp.float32) * (MAXR / (NB - 1))
    rbf = jnp.exp(-(((r - centers) / width) ** 2)) * np.sqrt(NB)
    cut = 0.5 * (jnp.cos(jnp.pi * jnp.clip(r / MAXR, 0.0, 1.0)) + 1.0)
    return rbf, cut


def _logits_body(xsa_ref, qa_ref, wk1_ref, wk2_ref, wklin_ref,
                 logits_ref, cut_ref, r2_ref):
    ps = xsa_ref[:, D:D + 3]
    pd = qa_ref[:, DK:DK + 3]
    ev = pd - ps
    r2 = jnp.sum(ev * ev, axis=1, keepdims=True)
    r2_ref[...] = r2
    rbf, cut = _rbf_cut(r2)
    cut_ref[...] = cut
    a = jax.nn.silu(jnp.dot(rbf, wk1_ref[...],
                            preferred_element_type=jnp.float32))
    wk = jnp.dot(a, wk2_ref[...], preferred_element_type=jnp.float32)
    k = jnp.dot(xsa_ref[:, :D] * wk, wklin_ref[...],
                preferred_element_type=jnp.float32)
    logits_ref[...] = jnp.sum(qa_ref[:, :DK] * k, axis=1,
                              keepdims=True) * (1.0 / np.sqrt(DK))


def _logits_stage(xsa, qa, Wk1, Wk2, Wk_lin):
    nblk = E // BE
    edge = lambda cols: pl.BlockSpec((BE, cols), lambda i: (i, 0))
    full = lambda a: pl.BlockSpec(a.shape, lambda i: (0, 0))
    return pl.pallas_call(
        _logits_body,
        grid=(nblk,),
        in_specs=[edge(DXA), edge(DQA), full(Wk1), full(Wk2),
                  full(Wk_lin)],
        out_specs=[edge(1), edge(1), edge(1)],
        out_shape=[jax.ShapeDtypeStruct((E, 1), jnp.float32),
                   jax.ShapeDtypeStruct((E, 1), jnp.float32),
                   jax.ShapeDtypeStruct((E, 1), jnp.float32)],
    )(xsa, qa, Wk1, Wk2, Wk_lin)


def _value_body(xsa_ref, r2_ref, w_ref, wv1_ref, wv2_ref, wvlin_ref, vw_ref):
    rbf, _ = _rbf_cut(r2_ref[...])
    b = jax.nn.silu(jnp.dot(rbf, wv1_ref[...],
                            preferred_element_type=jnp.float32))
    wv = jnp.dot(b, wv2_ref[...], preferred_element_type=jnp.float32)
    v = jnp.dot(xsa_ref[:, :D] * wv, wvlin_ref[...],
                preferred_element_type=jnp.float32)
    vw_ref[...] = v * w_ref[...]


def _value_stage(xsa, r2, w, Wv1, Wv2, Wv_lin):
    nblk = E // BE
    edge = lambda cols: pl.BlockSpec((BE, cols), lambda i: (i, 0))
    full = lambda a: pl.BlockSpec(a.shape, lambda i: (0, 0))
    return pl.pallas_call(
        _value_body,
        grid=(nblk,),
        in_specs=[edge(DXA), edge(1), edge(1), full(Wv1), full(Wv2),
                  full(Wv_lin)],
        out_specs=edge(D),
        out_shape=jax.ShapeDtypeStruct((E, D), jnp.float32),
    )(xsa, r2, w, Wv1, Wv2, Wv_lin)


def _sum_body(p_ref, o_ref):
    o_ref[...] = p_ref[0] + p_ref[1]


def _sum_stage(outp):
    nblk = 5
    rows = N // nblk
    return pl.pallas_call(
        _sum_body,
        grid=(nblk,),
        in_specs=[pl.BlockSpec((NC, rows, D), lambda i: (0, i, 0))],
        out_specs=pl.BlockSpec((rows, D), lambda i: (i, 0)),
        out_shape=jax.ShapeDtypeStruct((N, D), jnp.float32),
    )(outp)


def _gather_body(xa_hbm, qa_hbm, src_hbm, dst_hbm,
                 xsa_out, qa_out,
                 idx_s, idx_d, xs_buf, q_buf, sem):
    wid = _wid()

    def chunk(ci, carry):
        base = wid * EPW + ci * GC
        pltpu.sync_copy(src_hbm.at[pl.ds(base, GC)], idx_s)
        pltpu.sync_copy(dst_hbm.at[pl.ds(base, GC)], idx_d)
        pltpu.async_copy(xa_hbm.at[idx_s], xs_buf, sem).wait()
        pltpu.sync_copy(xs_buf, xsa_out.at[pl.ds(base, GC), :])
        pltpu.async_copy(qa_hbm.at[idx_d], q_buf, sem).wait()
        pltpu.sync_copy(q_buf, qa_out.at[pl.ds(base, GC), :])
        return carry

    lax.fori_loop(0, EPW // GC, chunk, 0)


def _gather_stage(xa, qa_table, src, dst):
    kern = pl.kernel(
        _gather_body,
        out_type=[jax.ShapeDtypeStruct((E, DXA), jnp.float32),
                  jax.ShapeDtypeStruct((E, DQA), jnp.float32)],
        mesh=_sc_mesh(),
        compiler_params=_SC_PARAMS,
        scratch_types=[pltpu.VMEM((GC,), jnp.int32),
                       pltpu.VMEM((GC,), jnp.int32),
                       pltpu.VMEM((GC, DXA), jnp.float32),
                       pltpu.VMEM((GC, DQA), jnp.float32),
                       pltpu.SemaphoreType.DMA],
    )
    return kern(xa, qa_table, src, dst)


def _z_body(log_hbm, cut_hbm, dst_hbm, zpart_out, e_out,
            z_loc, logb, cutb, dstb, eb, comb, zsl, stage):
    cid = lax.axis_index("c")
    sid = lax.axis_index("s")
    wid = sid * NC + cid

    def zinit(i, c):
        z_loc[pl.ds(i * L, L)] = jnp.zeros((L,), jnp.float32)
        return c

    lax.fori_loop(0, NP // L, zinit, 0)

    def chunk(ci, carry):
        base = wid * EPW + ci * CB
        pltpu.sync_copy(log_hbm.at[pl.ds(base, CB)], logb)
        pltpu.sync_copy(cut_hbm.at[pl.ds(base, CB)], cutb)
        pltpu.sync_copy(dst_hbm.at[pl.ds(base, CB)], dstb)

        def inner(j, c2):
            sl = pl.ds(j * L, L)
            d = dstb[sl]
            e = cutb[sl] * jnp.exp(logb[sl])
            plsc.addupdate_scatter(z_loc, [d], e)
            eb[sl] = e
            return c2

        lax.fori_loop(0, CB // L, inner, 0)
        pltpu.sync_copy(eb, e_out.at[pl.ds(base, CB)])
        return carry

    lax.fori_loop(0, EPW // CB, chunk, 0)

    pltpu.sync_copy(z_loc, stage.at[sid])
    plsc.subcore_barrier()
    off = sid * SL
    pltpu.sync_copy(stage.at[:, pl.ds(off, SL)], comb)

    def comb_loop(j, c):
        sl = pl.ds(j * L, L)
        s = comb[0, sl]
        for t in range(1, NS):
            s = s + comb[t, sl]
        zsl[sl] = s
        return c

    lax.fori_loop(0, SL // L, comb_loop, 0)
    pltpu.sync_copy(zsl, zpart_out.at[cid, pl.ds(off, SL)])


def _z_stage(logits, cut, dst):
    kern = pl.kernel(
        _z_body,
        out_type=[jax.ShapeDtypeStruct((NC, NP), jnp.float32),
                  jax.ShapeDtypeStruct((E,), jnp.float32)],
        mesh=_sc_mesh(),
        compiler_params=_SC_PARAMS,
        scratch_types=[pltpu.VMEM((NP,), jnp.float32),
                       pltpu.VMEM((CB,), jnp.float32),
                       pltpu.VMEM((CB,), jnp.float32),
                       pltpu.VMEM((CB,), jnp.int32),
                       pltpu.VMEM((CB,), jnp.float32),
                       pltpu.VMEM((NS, SL), jnp.float32),
                       pltpu.VMEM((SL,), jnp.float32),
                       pltpu.VMEM_SHARED((NS, NP), jnp.float32)],
    )
    return kern(logits, cut, dst)


def _sqrt16(x):
    i = plsc.bitcast(x, jnp.int32)
    y = plsc.bitcast((i >> 1) + 0x1FBD1DF6, jnp.float32)
    y = 0.5 * (y + x / y)
    y = 0.5 * (y + x / y)
    y = 0.5 * (y + x / y)
    return y


def _w_body(e_hbm, dst_hbm, zpart_hbm, w_out, zf, z1, eb, dstb, wb):
    wid = _wid()
    pltpu.sync_copy(zpart_hbm.at[0], zf)
    pltpu.sync_copy(zpart_hbm.at[1], z1)

    def zsum(i, c):
        sl = pl.ds(i * L, L)
        zf[sl] = zf[sl] + z1[sl] + 1e-9
        return c

    lax.fori_loop(0, NP // L, zsum, 0)

    def chunk(ci, carry):
        base = wid * EPW + ci * CB
        pltpu.sync_copy(e_hbm.at[pl.ds(base, CB)], eb)
        pltpu.sync_copy(dst_hbm.at[pl.ds(base, CB)], dstb)

        def inner(j, c2):
            sl = pl.ds(j * L, L)
            zg = plsc.load_gather(zf, [dstb[sl]])
            wb[sl] = _sqrt16(eb[sl] / zg + 1e-12)
            return c2

        lax.fori_loop(0, CB // L, inner, 0)
        pltpu.sync_copy(wb, w_out.at[pl.ds(base, CB)])
        return carry

    lax.fori_loop(0, EPW // CB, chunk, 0)


def _w_stage(e, dst, zpart):
    kern = pl.kernel(
        _w_body,
        out_type=jax.ShapeDtypeStruct((E,), jnp.float32),
        mesh=_sc_mesh(),
        compiler_params=_SC_PARAMS,
        scratch_types=[pltpu.VMEM((NP,), jnp.float32),
                       pltpu.VMEM((NP,), jnp.float32),
                       pltpu.VMEM((CB,), jnp.float32),
                       pltpu.VMEM((CB,), jnp.int32),
                       pltpu.VMEM((CB,), jnp.float32)],
    )
    return kern(e, dst, zpart)


GC2 = 200
ZR = 25


def _scatter_body(vw_hbm, dst_hbm, outp_out, acc, vbuf, idxb, zbuf):
    cid = lax.axis_index("c")
    sid = lax.axis_index("s")
    wid = sid * NC + cid

    def zloop(r, c):
        for k in range(D // L):
            zbuf[r, pl.ds(k * L, L)] = jnp.zeros((L,), jnp.float32)
        return c

    lax.fori_loop(0, ZR, zloop, 0)

    def zcopy(t, c):
        pltpu.sync_copy(zbuf, acc.at[pl.ds(sid * RT + t * ZR, ZR), :])
        return c

    lax.fori_loop(0, RT // ZR, zcopy, 0)
    plsc.subcore_barrier()

    def chunk(ci, carry):
        base = wid * EPW + ci * GC2
        pltpu.sync_copy(vw_hbm.at[pl.ds(base, GC2), :], vbuf)
        pltpu.sync_copy(dst_hbm.at[pl.ds(base, GC2)], idxb)
        pltpu.sync_copy(vbuf, acc.at[idxb], add=True)
        return carry

    lax.fori_loop(0, EPW // GC2, chunk, 0)
    plsc.subcore_barrier()
    pltpu.sync_copy(acc.at[pl.ds(sid * RT, RT), :],
                    outp_out.at[cid, pl.ds(sid * RT, RT), :])


def _scatter_stage(vw, dst):
    kern = pl.kernel(
        _scatter_body,
        out_type=jax.ShapeDtypeStruct((NC, N, D), jnp.float32),
        mesh=_sc_mesh(),
        compiler_params=_SC_PARAMS,
        scratch_types=[pltpu.VMEM_SHARED((N, D), jnp.float32),
                       pltpu.VMEM((GC2, D), jnp.float32),
                       pltpu.VMEM((GC2,), jnp.int32),
                       pltpu.VMEM((ZR, D), jnp.float32)],
    )
    return kern(vw, dst)


def kernel(x, pos, edge_index, Wq, Wk_lin, Wv_lin, Wk1, Wk2, Wv1, Wv2):
    src = edge_index[0]
    dst = edge_index[1]
    Q = _proj_stage(x, Wq)
    pospad = jnp.pad(pos, ((0, 0), (0, 13)))
    xa = jnp.concatenate([x, pospad], axis=1)
    qa_table = jnp.concatenate([Q, pospad], axis=1)
    xsa, qa = _gather_stage(xa, qa_table, src, dst)
    logits2, cut2, r2 = _logits_stage(xsa, qa, Wk1, Wk2, Wk_lin)
    zpart, e = _z_stage(logits2.reshape(E), cut2.reshape(E), dst)
    w = _w_stage(e, dst, zpart)
    vw = _value_stage(xsa, r2, w.reshape(E, 1), Wv1, Wv2, Wv_lin)
    outp = _scatter_stage(vw, dst)
    return _sum_stage(outp)


# speedup vs baseline: 5.9222x; 1.2789x over previous
import jax
import jax.numpy as jnp
import numpy as np
from jax import lax
from jax.experimental import pallas as pl
from jax.experimental.pallas import tpu as pltpu
from jax.experimental.pallas import tpu_sc as plsc

N = 10000
E = 320000
D = 128
DK = 32
NB = 10
MAXR = 3.0

NC = 2
NS = 16
NW = NC * NS
L = 16

NP = 10240
EPW = E // NW
GC = 400
CB = 2000
SL = NP // NS
RT = N // NS
BE = 2560


def _sc_mesh():
    return plsc.VectorSubcoreMesh(
        core_axis_name="c", subcore_axis_name="s",
        num_cores=NC, num_subcores=NS)


_SC_PARAMS = pltpu.CompilerParams(use_tc_tiling_on_sc=False,
                                  needs_layout_passes=False)


def _wid():
    return lax.axis_index("s") * NC + lax.axis_index("c")


def _proj_body(x_ref, wq_ref, q_ref):
    q_ref[...] = jnp.dot(x_ref[...], wq_ref[...],
                         preferred_element_type=jnp.float32)


def _proj_stage(x, Wq):
    return pl.pallas_call(
        _proj_body,
        out_shape=jax.ShapeDtypeStruct((N, DK), jnp.float32),
    )(x, Wq)


DXA = D + 16
DQA = DK + 16


RB = BE // 128
NL = E // 128


def _radial_body(ps_ref, pd_ref, wk1t_ref, wv1t_ref,
                 at_ref, bt_ref, cut_ref):
    psT = jnp.transpose(ps_ref[:, D:DXA], (1, 0)).reshape(16, RB, 128)
    pdT = jnp.transpose(pd_ref[:, DK:DQA], (1, 0)).reshape(16, RB, 128)
    ev = pdT - psT
    r2 = jnp.sum(ev * ev, axis=0)
    r = jnp.sqrt(r2 + 1e-9)
    width = MAXR / NB
    rbf = jnp.stack([
        jnp.exp(-(((r - (MAXR / (NB - 1)) * k) / width) ** 2))
        for k in range(NB)
    ]) * np.sqrt(NB)
    rbf2 = rbf.reshape(NB, BE)
    at_ref[...] = jax.nn.silu(jnp.dot(wk1t_ref[...], rbf2,
                                      preferred_element_type=jnp.float32))
    bt_ref[...] = jax.nn.silu(jnp.dot(wv1t_ref[...], rbf2,
                                      preferred_element_type=jnp.float32))
    cut = 0.5 * (jnp.cos(jnp.pi * jnp.clip(r / MAXR, 0.0, 1.0)) + 1.0)
    cut_ref[...] = cut.reshape(1, BE)


def _radial_stage(xsa, qa, Wk1_T, Wv1_T):
    nblk = E // BE
    full = lambda a: pl.BlockSpec(a.shape, lambda i: (0, 0))
    chan = pl.BlockSpec((16, BE), lambda i: (0, i))
    return pl.pallas_call(
        _radial_body,
        grid=(nblk,),
        in_specs=[pl.BlockSpec((BE, DXA), lambda i: (i, 0)),
                  pl.BlockSpec((BE, DQA), lambda i: (i, 0)),
                  full(Wk1_T), full(Wv1_T)],
        out_specs=[chan, chan, pl.BlockSpec((1, BE), lambda i: (0, i))],
        out_shape=[jax.ShapeDtypeStruct((16, E), jnp.float32),
                   jax.ShapeDtypeStruct((16, E), jnp.float32),
                   jax.ShapeDtypeStruct((1, E), jnp.float32)],
    )(xsa, qa, Wk1_T, Wv1_T)


def _logits_body(xsa_ref, qa_ref, at_ref, wklint_ref, wk2t_ref, logits_ref):
    u = jnp.dot(qa_ref[:, :DK], wklint_ref[...],
                preferred_element_type=jnp.float32)
    p = jnp.dot(u * xsa_ref[:, :D], wk2t_ref[...],
                preferred_element_type=jnp.float32)
    pT = jnp.transpose(p, (1, 0)).reshape(16, RB, 128)
    at = at_ref[...].reshape(16, RB, 128)
    acc = at[0] * pT[0]
    for j in range(1, 16):
        acc = acc + at[j] * pT[j]
    logits_ref[...] = acc.reshape(1, BE)


def _logits_stage(xsa, qa, at, Wk_lin_Ts, Wk2_T):
    nblk = E // BE
    edge = lambda cols: pl.BlockSpec((BE, cols), lambda i: (i, 0))
    full = lambda a: pl.BlockSpec(a.shape, lambda i: (0, 0))
    return pl.pallas_call(
        _logits_body,
        grid=(nblk,),
        in_specs=[edge(DXA), edge(DQA),
                  pl.BlockSpec((16, BE), lambda i: (0, i)),
                  full(Wk_lin_Ts), full(Wk2_T)],
        out_specs=pl.BlockSpec((1, BE), lambda i: (0, i)),
        out_shape=jax.ShapeDtypeStruct((1, E), jnp.float32),
    )(xsa, qa, at, Wk_lin_Ts, Wk2_T)


def _value_body(xsa_ref, bt_ref, w_ref, wv2_ref, wvlin_ref, vw_ref):
    b = jnp.transpose(bt_ref[...], (1, 0))
    wv = jnp.dot(b, wv2_ref[...], preferred_element_type=jnp.float32)
    v = jnp.dot(xsa_ref[:, :D] * wv, wvlin_ref[...],
                preferred_element_type=jnp.float32)
    vw_ref[...] = v * w_ref[...]


def _value_stage(xsa, bt, w, Wv2, Wv_lin):
    nblk = E // BE
    edge = lambda cols: pl.BlockSpec((BE, cols), lambda i: (i, 0))
    full = lambda a: pl.BlockSpec(a.shape, lambda i: (0, 0))
    return pl.pallas_call(
        _value_body,
        grid=(nblk,),
        in_specs=[edge(DXA),
                  pl.BlockSpec((16, BE), lambda i: (0, i)),
                  edge(1), full(Wv2), full(Wv_lin)],
        out_specs=edge(D),
        out_shape=jax.ShapeDtypeStruct((E, D), jnp.float32),
    )(xsa, bt, w, Wv2, Wv_lin)


def _sum_body(p_ref, o_ref):
    o_ref[...] = p_ref[0] + p_ref[1]


def _sum_stage(outp):
    nblk = 5
    rows = N // nblk
    return pl.pallas_call(
        _sum_body,
        grid=(nblk,),
        in_specs=[pl.BlockSpec((NC, rows, D), lambda i: (0, i, 0))],
        out_specs=pl.BlockSpec((rows, D), lambda i: (i, 0)),
        out_shape=jax.ShapeDtypeStruct((N, D), jnp.float32),
    )(outp)


def _gather_body(xa_hbm, qa_hbm, src_hbm, dst_hbm,
                 xsa_out, qa_out,
                 idx_s, idx_d, xs_buf, q_buf, sem):
    wid = _wid()

    def chunk(ci, carry):
        base = wid * EPW + ci * GC
        pltpu.sync_copy(src_hbm.at[pl.ds(base, GC)], idx_s)
        pltpu.sync_copy(dst_hbm.at[pl.ds(base, GC)], idx_d)
        pltpu.async_copy(xa_hbm.at[idx_s], xs_buf, sem).wait()
        pltpu.sync_copy(xs_buf, xsa_out.at[pl.ds(base, GC), :])
        pltpu.async_copy(qa_hbm.at[idx_d], q_buf, sem).wait()
        pltpu.sync_copy(q_buf, qa_out.at[pl.ds(base, GC), :])
        return carry

    lax.fori_loop(0, EPW // GC, chunk, 0)


def _gather_stage(xa, qa_table, src, dst):
    kern = pl.kernel(
        _gather_body,
        out_type=[jax.ShapeDtypeStruct((E, DXA), jnp.float32),
                  jax.ShapeDtypeStruct((E, DQA), jnp.float32)],
        mesh=_sc_mesh(),
        compiler_params=_SC_PARAMS,
        scratch_types=[pltpu.VMEM((GC,), jnp.int32),
                       pltpu.VMEM((GC,), jnp.int32),
                       pltpu.VMEM((GC, DXA), jnp.float32),
                       pltpu.VMEM((GC, DQA), jnp.float32),
                       pltpu.SemaphoreType.DMA],
    )
    return kern(xa, qa_table, src, dst)


def _z_body(log_hbm, cut_hbm, dst_hbm, zpart_out, e_out,
            z_loc, logb, cutb, dstb, eb, comb, zsl, stage):
    cid = lax.axis_index("c")
    sid = lax.axis_index("s")
    wid = sid * NC + cid

    def zinit(i, c):
        z_loc[pl.ds(i * L, L)] = jnp.zeros((L,), jnp.float32)
        return c

    lax.fori_loop(0, NP // L, zinit, 0)

    def chunk(ci, carry):
        base = wid * EPW + ci * CB
        pltpu.sync_copy(log_hbm.at[pl.ds(base, CB)], logb)
        pltpu.sync_copy(cut_hbm.at[pl.ds(base, CB)], cutb)
        pltpu.sync_copy(dst_hbm.at[pl.ds(base, CB)], dstb)

        def inner(j, c2):
            sl = pl.ds(j * L, L)
            d = dstb[sl]
            e = cutb[sl] * jnp.exp(logb[sl])
            plsc.addupdate_scatter(z_loc, [d], e)
            eb[sl] = e
            return c2

        lax.fori_loop(0, CB // L, inner, 0)
        pltpu.sync_copy(eb, e_out.at[pl.ds(base, CB)])
        return carry

    lax.fori_loop(0, EPW // CB, chunk, 0)

    pltpu.sync_copy(z_loc, stage.at[sid])
    plsc.subcore_barrier()
    off = sid * SL
    pltpu.sync_copy(stage.at[:, pl.ds(off, SL)], comb)

    def comb_loop(j, c):
        sl = pl.ds(j * L, L)
        s = comb[0, sl]
        for t in range(1, NS):
            s = s + comb[t, sl]
        zsl[sl] = s
        return c

    lax.fori_loop(0, SL // L, comb_loop, 0)
    pltpu.sync_copy(zsl, zpart_out.at[cid, pl.ds(off, SL)])


def _z_stage(logits, cut, dst):
    kern = pl.kernel(
        _z_body,
        out_type=[jax.ShapeDtypeStruct((NC, NP), jnp.float32),
                  jax.ShapeDtypeStruct((E,), jnp.float32)],
        mesh=_sc_mesh(),
        compiler_params=_SC_PARAMS,
        scratch_types=[pltpu.VMEM((NP,), jnp.float32),
                       pltpu.VMEM((CB,), jnp.float32),
                       pltpu.VMEM((CB,), jnp.float32),
                       pltpu.VMEM((CB,), jnp.int32),
                       pltpu.VMEM((CB,), jnp.float32),
                       pltpu.VMEM((NS, SL), jnp.float32),
                       pltpu.VMEM((SL,), jnp.float32),
                       pltpu.VMEM_SHARED((NS, NP), jnp.float32)],
    )
    return kern(logits, cut, dst)


def _sqrt16(x):
    i = plsc.bitcast(x, jnp.int32)
    y = plsc.bitcast((i >> 1) + 0x1FBD1DF6, jnp.float32)
    y = 0.5 * (y + x / y)
    y = 0.5 * (y + x / y)
    y = 0.5 * (y + x / y)
    return y


def _w_body(e_hbm, dst_hbm, zpart_hbm, w_out, zf, z1, eb, dstb, wb):
    wid = _wid()
    pltpu.sync_copy(zpart_hbm.at[0], zf)
    pltpu.sync_copy(zpart_hbm.at[1], z1)

    def zsum(i, c):
        sl = pl.ds(i * L, L)
        zf[sl] = zf[sl] + z1[sl] + 1e-9
        return c

    lax.fori_loop(0, NP // L, zsum, 0)

    def chunk(ci, carry):
        base = wid * EPW + ci * CB
        pltpu.sync_copy(e_hbm.at[pl.ds(base, CB)], eb)
        pltpu.sync_copy(dst_hbm.at[pl.ds(base, CB)], dstb)

        def inner(j, c2):
            sl = pl.ds(j * L, L)
            zg = plsc.load_gather(zf, [dstb[sl]])
            wb[sl] = _sqrt16(eb[sl] / zg + 1e-12)
            return c2

        lax.fori_loop(0, CB // L, inner, 0)
        pltpu.sync_copy(wb, w_out.at[pl.ds(base, CB)])
        return carry

    lax.fori_loop(0, EPW // CB, chunk, 0)


def _w_stage(e, dst, zpart):
    kern = pl.kernel(
        _w_body,
        out_type=jax.ShapeDtypeStruct((E,), jnp.float32),
        mesh=_sc_mesh(),
        compiler_params=_SC_PARAMS,
        scratch_types=[pltpu.VMEM((NP,), jnp.float32),
                       pltpu.VMEM((NP,), jnp.float32),
                       pltpu.VMEM((CB,), jnp.float32),
                       pltpu.VMEM((CB,), jnp.int32),
                       pltpu.VMEM((CB,), jnp.float32)],
    )
    return kern(e, dst, zpart)


GC2 = 200
ZR = 25


def _scatter_body(vw_hbm, dst_hbm, outp_out, acc, vbuf, idxb, zbuf):
    cid = lax.axis_index("c")
    sid = lax.axis_index("s")
    wid = sid * NC + cid

    def zloop(r, c):
        for k in range(D // L):
            zbuf[r, pl.ds(k * L, L)] = jnp.zeros((L,), jnp.float32)
        return c

    lax.fori_loop(0, ZR, zloop, 0)

    def zcopy(t, c):
        pltpu.sync_copy(zbuf, acc.at[pl.ds(sid * RT + t * ZR, ZR), :])
        return c

    lax.fori_loop(0, RT // ZR, zcopy, 0)
    plsc.subcore_barrier()

    def chunk(ci, carry):
        base = wid * EPW + ci * GC2
        pltpu.sync_copy(vw_hbm.at[pl.ds(base, GC2), :], vbuf)
        pltpu.sync_copy(dst_hbm.at[pl.ds(base, GC2)], idxb)
        pltpu.sync_copy(vbuf, acc.at[idxb], add=True)
        return carry

    lax.fori_loop(0, EPW // GC2, chunk, 0)
    plsc.subcore_barrier()
    pltpu.sync_copy(acc.at[pl.ds(sid * RT, RT), :],
                    outp_out.at[cid, pl.ds(sid * RT, RT), :])


def _scatter_stage(vw, dst):
    kern = pl.kernel(
        _scatter_body,
        out_type=jax.ShapeDtypeStruct((NC, N, D), jnp.float32),
        mesh=_sc_mesh(),
        compiler_params=_SC_PARAMS,
        scratch_types=[pltpu.VMEM_SHARED((N, D), jnp.float32),
                       pltpu.VMEM((GC2, D), jnp.float32),
                       pltpu.VMEM((GC2,), jnp.int32),
                       pltpu.VMEM((ZR, D), jnp.float32)],
    )
    return kern(vw, dst)


def kernel(x, pos, edge_index, Wq, Wk_lin, Wv_lin, Wk1, Wk2, Wv1, Wv2):
    src = edge_index[0]
    dst = edge_index[1]
    Q = _proj_stage(x, Wq)
    pospad = jnp.pad(pos, ((0, 0), (0, 13)))
    xa = jnp.concatenate([x, pospad], axis=1)
    qa_table = jnp.concatenate([Q, pospad], axis=1)
    xsa, qa = _gather_stage(xa, qa_table, src, dst)
    at, bt, cut_ld = _radial_stage(xsa, qa, Wk1.T, Wv1.T)
    logits_ld = _logits_stage(xsa, qa, at,
                              Wk_lin.T * (1.0 / np.sqrt(DK)), Wk2.T)
    zpart, e = _z_stage(logits_ld.reshape(E), cut_ld.reshape(E), dst)
    w = _w_stage(e, dst, zpart)
    vw = _value_stage(xsa, bt, w.reshape(E, 1), Wv2, Wv_lin)
    outp = _scatter_stage(vw, dst)
    return _sum_stage(outp)


# speedup vs baseline: 7.0153x; 1.1846x over previous
import jax
import jax.numpy as jnp
import numpy as np
from jax import lax
from jax.experimental import pallas as pl
from jax.experimental.pallas import tpu as pltpu
from jax.experimental.pallas import tpu_sc as plsc

N = 10000
E = 320000
D = 128
DK = 32
NB = 10
MAXR = 3.0

NC = 2
NS = 16
NW = NC * NS
L = 16

NP = 10240
EPW = E // NW
GC = 400
CB = 2000
SL = NP // NS
RT = N // NS
BE = 2560


def _sc_mesh():
    return plsc.VectorSubcoreMesh(
        core_axis_name="c", subcore_axis_name="s",
        num_cores=NC, num_subcores=NS)


_SC_PARAMS = pltpu.CompilerParams(use_tc_tiling_on_sc=False,
                                  needs_layout_passes=False)


def _wid():
    return lax.axis_index("s") * NC + lax.axis_index("c")


def _proj_body(x_ref, wq_ref, q_ref):
    q_ref[...] = jnp.dot(x_ref[...], wq_ref[...],
                         preferred_element_type=jnp.float32)


def _proj_stage(x, Wq):
    return pl.pallas_call(
        _proj_body,
        out_shape=jax.ShapeDtypeStruct((N, DK), jnp.float32),
    )(x, Wq)


DXA = D + 16
DQA = DK + 16


RB = BE // 128
NL = E // 128


def _edge_body(xsa_ref, qa_ref, wk1t_ref, wv1t_ref, wklint_ref,
               wk2t_ref, bt_ref, se_ref):
    psT = jnp.transpose(xsa_ref[:, D:DXA], (1, 0)).reshape(16, RB, 128)
    pdT = jnp.transpose(qa_ref[:, DK:DQA], (1, 0)).reshape(16, RB, 128)
    ev = pdT - psT
    r2 = jnp.sum(ev * ev, axis=0)
    r = jnp.sqrt(r2 + 1e-9)
    width = MAXR / NB
    rbf = jnp.stack([
        jnp.exp(-(((r - (MAXR / (NB - 1)) * k) / width) ** 2))
        for k in range(NB)
    ]) * np.sqrt(NB)
    rbf2 = rbf.reshape(NB, BE)
    at = jax.nn.silu(jnp.dot(wk1t_ref[...], rbf2,
                             preferred_element_type=jnp.float32))
    bt_ref[...] = jax.nn.silu(jnp.dot(wv1t_ref[...], rbf2,
                                      preferred_element_type=jnp.float32))
    cut = 0.5 * (jnp.cos(jnp.pi * jnp.clip(r / MAXR, 0.0, 1.0)) + 1.0)

    u = jnp.dot(qa_ref[:, :DK], wklint_ref[...],
                preferred_element_type=jnp.float32)
    p = jnp.dot(u * xsa_ref[:, :D], wk2t_ref[...],
                preferred_element_type=jnp.float32)
    pT = jnp.transpose(p, (1, 0)).reshape(16, RB, 128)
    at3 = at.reshape(16, RB, 128)
    acc = at3[0] * pT[0]
    for j in range(1, 16):
        acc = acc + at3[j] * pT[j]
    se = jnp.sqrt(cut * jnp.exp(acc))
    se_ref[...] = se.reshape(1, BE)


def _edge_stage(xsa, qa, Wk1_T, Wv1_T, Wk_lin_Ts, Wk2_T):
    nblk = E // BE
    full = lambda a: pl.BlockSpec(a.shape, lambda i: (0, 0))
    return pl.pallas_call(
        _edge_body,
        grid=(nblk,),
        in_specs=[pl.BlockSpec((BE, DXA), lambda i: (i, 0)),
                  pl.BlockSpec((BE, DQA), lambda i: (i, 0)),
                  full(Wk1_T), full(Wv1_T), full(Wk_lin_Ts), full(Wk2_T)],
        out_specs=[pl.BlockSpec((16, BE), lambda i: (0, i)),
                   pl.BlockSpec((1, BE), lambda i: (0, i))],
        out_shape=[jax.ShapeDtypeStruct((16, E), jnp.float32),
                   jax.ShapeDtypeStruct((1, E), jnp.float32)],
    )(xsa, qa, Wk1_T, Wv1_T, Wk_lin_Ts, Wk2_T)


def _value_body(xsa_ref, bt_ref, se_ref, wv2_ref, wvlin_ref, vw_ref):
    b = jnp.transpose(bt_ref[...], (1, 0))
    wv = jnp.dot(b, wv2_ref[...], preferred_element_type=jnp.float32)
    v = jnp.dot(xsa_ref[:, :D] * wv, wvlin_ref[...],
                preferred_element_type=jnp.float32)
    vw_ref[...] = v * se_ref[...]


def _value_stage(xsa, bt, se, Wv2, Wv_lin):
    nblk = E // BE
    edge = lambda cols: pl.BlockSpec((BE, cols), lambda i: (i, 0))
    full = lambda a: pl.BlockSpec(a.shape, lambda i: (0, 0))
    return pl.pallas_call(
        _value_body,
        grid=(nblk,),
        in_specs=[edge(DXA),
                  pl.BlockSpec((16, BE), lambda i: (0, i)),
                  edge(1), full(Wv2), full(Wv_lin)],
        out_specs=edge(D),
        out_shape=jax.ShapeDtypeStruct((E, D), jnp.float32),
    )(xsa, bt, se, Wv2, Wv_lin)


def _finish_body(p_ref, ztp_ref, o_ref):
    zsum = ztp_ref[:, 0:1] + ztp_ref[:, 1:2] + 1e-9
    o_ref[...] = (p_ref[0] + p_ref[1]) * lax.rsqrt(zsum)


def _finish_stage(outp, zpartT):
    nblk = 5
    rows = N // nblk
    return pl.pallas_call(
        _finish_body,
        grid=(nblk,),
        in_specs=[pl.BlockSpec((NC, rows, D), lambda i: (0, i, 0)),
                  pl.BlockSpec((rows, NC), lambda i: (i, 0))],
        out_specs=pl.BlockSpec((rows, D), lambda i: (i, 0)),
        out_shape=jax.ShapeDtypeStruct((N, D), jnp.float32),
    )(outp, zpartT)


def _gather_body(xa_hbm, qa_hbm, src_hbm, dst_hbm,
                 xsa_out, qa_out,
                 idx_s, idx_d, xs_buf, q_buf, sem):
    wid = _wid()

    def chunk(ci, carry):
        base = wid * EPW + ci * GC
        pltpu.sync_copy(src_hbm.at[pl.ds(base, GC)], idx_s)
        pltpu.sync_copy(dst_hbm.at[pl.ds(base, GC)], idx_d)
        pltpu.async_copy(xa_hbm.at[idx_s], xs_buf, sem).wait()
        pltpu.sync_copy(xs_buf, xsa_out.at[pl.ds(base, GC), :])
        pltpu.async_copy(qa_hbm.at[idx_d], q_buf, sem).wait()
        pltpu.sync_copy(q_buf, qa_out.at[pl.ds(base, GC), :])
        return carry

    lax.fori_loop(0, EPW // GC, chunk, 0)


def _gather_stage(xa, qa_table, src, dst):
    kern = pl.kernel(
        _gather_body,
        out_type=[jax.ShapeDtypeStruct((E, DXA), jnp.float32),
                  jax.ShapeDtypeStruct((E, DQA), jnp.float32)],
        mesh=_sc_mesh(),
        compiler_params=_SC_PARAMS,
        scratch_types=[pltpu.VMEM((GC,), jnp.int32),
                       pltpu.VMEM((GC,), jnp.int32),
                       pltpu.VMEM((GC, DXA), jnp.float32),
                       pltpu.VMEM((GC, DQA), jnp.float32),
                       pltpu.SemaphoreType.DMA],
    )
    return kern(xa, qa_table, src, dst)


def _z_body(se_hbm, dst_hbm, zpart_out,
            z_loc, seb, dstb, comb, zsl, stage):
    cid = lax.axis_index("c")
    sid = lax.axis_index("s")
    wid = sid * NC + cid

    def zinit(i, c):
        z_loc[pl.ds(i * L, L)] = jnp.zeros((L,), jnp.float32)
        return c

    lax.fori_loop(0, NP // L, zinit, 0)

    def chunk(ci, carry):
        base = wid * EPW + ci * CB
        pltpu.sync_copy(se_hbm.at[pl.ds(base, CB)], seb)
        pltpu.sync_copy(dst_hbm.at[pl.ds(base, CB)], dstb)

        def inner(j, c2):
            sl = pl.ds(j * L, L)
            sev = seb[sl]
            plsc.addupdate_scatter(z_loc, [dstb[sl]], sev * sev)
            return c2

        lax.fori_loop(0, CB // L, inner, 0)
        return carry

    lax.fori_loop(0, EPW // CB, chunk, 0)

    pltpu.sync_copy(z_loc, stage.at[sid])
    plsc.subcore_barrier()
    off = sid * SL
    pltpu.sync_copy(stage.at[:, pl.ds(off, SL)], comb)

    def comb_loop(j, c):
        sl = pl.ds(j * L, L)
        s = comb[0, sl]
        for t in range(1, NS):
            s = s + comb[t, sl]
        zsl[sl] = s
        return c

    lax.fori_loop(0, SL // L, comb_loop, 0)
    pltpu.sync_copy(zsl, zpart_out.at[cid, pl.ds(off, SL)])


def _z_stage(se, dst):
    kern = pl.kernel(
        _z_body,
        out_type=jax.ShapeDtypeStruct((NC, NP), jnp.float32),
        mesh=_sc_mesh(),
        compiler_params=_SC_PARAMS,
        scratch_types=[pltpu.VMEM((NP,), jnp.float32),
                       pltpu.VMEM((CB,), jnp.float32),
                       pltpu.VMEM((CB,), jnp.int32),
                       pltpu.VMEM((NS, SL), jnp.float32),
                       pltpu.VMEM((SL,), jnp.float32),
                       pltpu.VMEM_SHARED((NS, NP), jnp.float32)],
    )
    return kern(se, dst)


GC2 = 200
ZR = 25


def _scatter_body(vw_hbm, dst_hbm, outp_out, acc, vbuf, idxb, zbuf):
    cid = lax.axis_index("c")
    sid = lax.axis_index("s")
    wid = sid * NC + cid

    def zloop(r, c):
        for k in range(D // L):
            zbuf[r, pl.ds(k * L, L)] = jnp.zeros((L,), jnp.float32)
        return c

    lax.fori_loop(0, ZR, zloop, 0)

    def zcopy(t, c):
        pltpu.sync_copy(zbuf, acc.at[pl.ds(sid * RT + t * ZR, ZR), :])
        return c

    lax.fori_loop(0, RT // ZR, zcopy, 0)
    plsc.subcore_barrier()

    def chunk(ci, carry):
        base = wid * EPW + ci * GC2
        pltpu.sync_copy(vw_hbm.at[pl.ds(base, GC2), :], vbuf)
        pltpu.sync_copy(dst_hbm.at[pl.ds(base, GC2)], idxb)
        pltpu.sync_copy(vbuf, acc.at[idxb], add=True)
        return carry

    lax.fori_loop(0, EPW // GC2, chunk, 0)
    plsc.subcore_barrier()
    pltpu.sync_copy(acc.at[pl.ds(sid * RT, RT), :],
                    outp_out.at[cid, pl.ds(sid * RT, RT), :])


def _scatter_stage(vw, dst):
    kern = pl.kernel(
        _scatter_body,
        out_type=jax.ShapeDtypeStruct((NC, N, D), jnp.float32),
        mesh=_sc_mesh(),
        compiler_params=_SC_PARAMS,
        scratch_types=[pltpu.VMEM_SHARED((N, D), jnp.float32),
                       pltpu.VMEM((GC2, D), jnp.float32),
                       pltpu.VMEM((GC2,), jnp.int32),
                       pltpu.VMEM((ZR, D), jnp.float32)],
    )
    return kern(vw, dst)


def kernel(x, pos, edge_index, Wq, Wk_lin, Wv_lin, Wk1, Wk2, Wv1, Wv2):
    src = edge_index[0]
    dst = edge_index[1]
    Q = _proj_stage(x, Wq)
    pospad = jnp.pad(pos, ((0, 0), (0, 13)))
    xa = jnp.concatenate([x, pospad], axis=1)
    qa_table = jnp.concatenate([Q, pospad], axis=1)
    xsa, qa = _gather_stage(xa, qa_table, src, dst)
    bt, se = _edge_stage(xsa, qa, Wk1.T, Wv1.T,
                         Wk_lin.T * (1.0 / np.sqrt(DK)), Wk2.T)
    zpartT = _z_stage(se.reshape(E), dst)
    vw = _value_stage(xsa, bt, se.reshape(E, 1), Wv2, Wv_lin)
    outp = _scatter_stage(vw, dst)
    return _finish_stage(outp, zpartT.T[:N])


# speedup vs baseline: 7.8100x; 1.1133x over previous
import jax
import jax.numpy as jnp
import numpy as np
from jax import lax
from jax.experimental import pallas as pl
from jax.experimental.pallas import tpu as pltpu
from jax.experimental.pallas import tpu_sc as plsc

N = 10000
E = 320000
D = 128
DK = 32
NB = 10
MAXR = 3.0

NC = 2
NS = 16
NW = NC * NS
L = 16

NP = 10240
EPW = E // NW
GC = 400
CB = 2000
SL = NP // NS
RT = N // NS
BE = 2560


def _sc_mesh():
    return plsc.VectorSubcoreMesh(
        core_axis_name="c", subcore_axis_name="s",
        num_cores=NC, num_subcores=NS)


_SC_PARAMS = pltpu.CompilerParams(use_tc_tiling_on_sc=False,
                                  needs_layout_passes=False)


def _wid():
    return lax.axis_index("s") * NC + lax.axis_index("c")


def _proj_body(x_ref, pos_ref, wq_ref, xa_ref, qa_ref):
    xa_ref[:, :D] = x_ref[...]
    xa_ref[:, D:] = pos_ref[...]
    qa_ref[:, :DK] = jnp.dot(x_ref[...], wq_ref[...],
                             preferred_element_type=jnp.float32)
    qa_ref[:, DK:] = pos_ref[...]


def _proj_stage(x, pospad, Wq):
    return pl.pallas_call(
        _proj_body,
        out_shape=[jax.ShapeDtypeStruct((N, DXA), jnp.float32),
                   jax.ShapeDtypeStruct((N, DQA), jnp.float32)],
    )(x, pospad, Wq)


DXA = D + 16
DQA = DK + 16


RB = BE // 128
NL = E // 128


def _edge_body(xsa_ref, qa_ref, wk1t_ref, wv1t_ref, wklint_ref,
               wk2t_ref, wv2_ref, wvlin_ref, se_ref, vw_ref):
    psT = jnp.transpose(xsa_ref[:, D:DXA], (1, 0)).reshape(16, RB, 128)
    pdT = jnp.transpose(qa_ref[:, DK:DQA], (1, 0)).reshape(16, RB, 128)
    ev = pdT - psT
    r2 = jnp.sum(ev * ev, axis=0)
    r = jnp.sqrt(r2 + 1e-9)
    width = MAXR / NB
    rbf = jnp.stack([
        jnp.exp(-(((r - (MAXR / (NB - 1)) * k) / width) ** 2))
        for k in range(NB)
    ]) * np.sqrt(NB)
    rbf2 = rbf.reshape(NB, BE)
    at = jax.nn.silu(jnp.dot(wk1t_ref[...], rbf2,
                             preferred_element_type=jnp.float32))
    bt = jax.nn.silu(jnp.dot(wv1t_ref[...], rbf2,
                             preferred_element_type=jnp.float32))
    cut = 0.5 * (jnp.cos(jnp.pi * jnp.clip(r / MAXR, 0.0, 1.0)) + 1.0)

    u = jnp.dot(qa_ref[:, :DK], wklint_ref[...],
                preferred_element_type=jnp.float32)
    xs = xsa_ref[:, :D]
    p = jnp.dot(u * xs, wk2t_ref[...],
                preferred_element_type=jnp.float32)
    pT = jnp.transpose(p, (1, 0)).reshape(16, RB, 128)
    at3 = at.reshape(16, RB, 128)
    acc = at3[0] * pT[0]
    for j in range(1, 16):
        acc = acc + at3[j] * pT[j]
    se = jnp.sqrt(cut * jnp.exp(acc))
    se_ref[...] = se.reshape(1, BE)

    b = jnp.transpose(bt, (1, 0))
    wv = jnp.dot(b, wv2_ref[...], preferred_element_type=jnp.float32)
    v = jnp.dot(xs * wv, wvlin_ref[...],
                preferred_element_type=jnp.float32)
    vw_ref[...] = v * jnp.transpose(se.reshape(1, BE), (1, 0))


def _edge_stage(xsa, qa, Wk1_T, Wv1_T, Wk_lin_Ts, Wk2_T, Wv2, Wv_lin):
    nblk = E // BE
    full = lambda a: pl.BlockSpec(a.shape, lambda i: (0, 0))
    return pl.pallas_call(
        _edge_body,
        grid=(nblk,),
        in_specs=[pl.BlockSpec((BE, DXA), lambda i: (i, 0)),
                  pl.BlockSpec((BE, DQA), lambda i: (i, 0)),
                  full(Wk1_T), full(Wv1_T), full(Wk_lin_Ts), full(Wk2_T),
                  full(Wv2), full(Wv_lin)],
        out_specs=[pl.BlockSpec((1, BE), lambda i: (0, i)),
                   pl.BlockSpec((BE, D), lambda i: (i, 0))],
        out_shape=[jax.ShapeDtypeStruct((1, E), jnp.float32),
                   jax.ShapeDtypeStruct((E, D), jnp.float32)],
    )(xsa, qa, Wk1_T, Wv1_T, Wk_lin_Ts, Wk2_T, Wv2, Wv_lin)


def _finish_body(p_ref, ztp_ref, o_ref):
    zsum = jnp.sum(ztp_ref[...], axis=1, keepdims=True) + 1e-9
    o_ref[...] = (p_ref[0] + p_ref[1]) * lax.rsqrt(zsum)


def _finish_stage(outp, zpartT):
    nblk = 5
    rows = N // nblk
    return pl.pallas_call(
        _finish_body,
        grid=(nblk,),
        in_specs=[pl.BlockSpec((NC, rows, D), lambda i: (0, i, 0)),
                  pl.BlockSpec((rows, NW), lambda i: (i, 0))],
        out_specs=pl.BlockSpec((rows, D), lambda i: (i, 0)),
        out_shape=jax.ShapeDtypeStruct((N, D), jnp.float32),
    )(outp, zpartT)


def _gather_body(xa_hbm, qa_hbm, src_hbm, dst_hbm,
                 xsa_out, qa_out,
                 idx_s, idx_d, xs_buf, q_buf, sem):
    wid = _wid()

    def chunk(ci, carry):
        base = wid * EPW + ci * GC
        pltpu.sync_copy(src_hbm.at[pl.ds(base, GC)], idx_s)
        pltpu.sync_copy(dst_hbm.at[pl.ds(base, GC)], idx_d)
        pltpu.async_copy(xa_hbm.at[idx_s], xs_buf, sem).wait()
        pltpu.sync_copy(xs_buf, xsa_out.at[pl.ds(base, GC), :])
        pltpu.async_copy(qa_hbm.at[idx_d], q_buf, sem).wait()
        pltpu.sync_copy(q_buf, qa_out.at[pl.ds(base, GC), :])
        return carry

    lax.fori_loop(0, EPW // GC, chunk, 0)


def _gather_stage(xa, qa_table, src, dst):
    kern = pl.kernel(
        _gather_body,
        out_type=[jax.ShapeDtypeStruct((E, DXA), jnp.float32),
                  jax.ShapeDtypeStruct((E, DQA), jnp.float32)],
        mesh=_sc_mesh(),
        compiler_params=_SC_PARAMS,
        scratch_types=[pltpu.VMEM((GC,), jnp.int32),
                       pltpu.VMEM((GC,), jnp.int32),
                       pltpu.VMEM((GC, DXA), jnp.float32),
                       pltpu.VMEM((GC, DQA), jnp.float32),
                       pltpu.SemaphoreType.DMA],
    )
    return kern(xa, qa_table, src, dst)


GC2 = 200
ZR = 25


def _scatter_body(vw_hbm, se_hbm, dst_hbm, outp_out, zpart_out,
                  acc, vbuf, idxb, z_loc, seb, dstb):
    cid = lax.axis_index("c")
    sid = lax.axis_index("s")
    wid = sid * NC + cid

    def zinit(i, c):
        z_loc[pl.ds(i * L, L)] = jnp.zeros((L,), jnp.float32)
        return c

    lax.fori_loop(0, NP // L, zinit, 0)

    def zloop(r, c):
        for k in range(D // L):
            vbuf[r, pl.ds(k * L, L)] = jnp.zeros((L,), jnp.float32)
        return c

    lax.fori_loop(0, ZR, zloop, 0)

    def zcopy(t, c):
        pltpu.sync_copy(vbuf.at[pl.ds(0, ZR), :],
                        acc.at[pl.ds(sid * RT + t * ZR, ZR), :])
        return c

    lax.fori_loop(0, RT // ZR, zcopy, 0)
    plsc.subcore_barrier()

    def zchunk(ci, carry):
        base = wid * EPW + ci * CB
        pltpu.sync_copy(se_hbm.at[pl.ds(base, CB)], seb)
        pltpu.sync_copy(dst_hbm.at[pl.ds(base, CB)], dstb)

        def inner(j, c2):
            sl = pl.ds(j * L, L)
            sev = seb[sl]
            plsc.addupdate_scatter(z_loc, [dstb[sl]], sev * sev)
            return c2

        lax.fori_loop(0, CB // L, inner, 0)
        return carry

    lax.fori_loop(0, EPW // CB, zchunk, 0)

    def chunk(ci, carry):
        base = wid * EPW + ci * GC2
        pltpu.sync_copy(vw_hbm.at[pl.ds(base, GC2), :], vbuf)
        pltpu.sync_copy(dst_hbm.at[pl.ds(base, GC2)], idxb)
        pltpu.sync_copy(vbuf, acc.at[idxb], add=True)
        return carry

    lax.fori_loop(0, EPW // GC2, chunk, 0)
    plsc.subcore_barrier()
    pltpu.sync_copy(acc.at[pl.ds(sid * RT, RT), :],
                    outp_out.at[cid, pl.ds(sid * RT, RT), :])
    pltpu.sync_copy(z_loc, zpart_out.at[wid])


def _scatter_stage(vw, se, dst):
    kern = pl.kernel(
        _scatter_body,
        out_type=[jax.ShapeDtypeStruct((NC, N, D), jnp.float32),
                  jax.ShapeDtypeStruct((NW, NP), jnp.float32)],
        mesh=_sc_mesh(),
        compiler_params=_SC_PARAMS,
        scratch_types=[pltpu.VMEM_SHARED((N, D), jnp.float32),
                       pltpu.VMEM((GC2, D), jnp.float32),
                       pltpu.VMEM((GC2,), jnp.int32),
                       pltpu.VMEM((NP,), jnp.float32),
                       pltpu.VMEM((CB,), jnp.float32),
                       pltpu.VMEM((CB,), jnp.int32)],
    )
    return kern(vw, se, dst)


def kernel(x, pos, edge_index, Wq, Wk_lin, Wv_lin, Wk1, Wk2, Wv1, Wv2):
    src = edge_index[0]
    dst = edge_index[1]
    pospad = jnp.pad(pos, ((0, 0), (0, 13)))
    xa, qa_table = _proj_stage(x, pospad, Wq)
    xsa, qa = _gather_stage(xa, qa_table, src, dst)
    se, vw = _edge_stage(xsa, qa, Wk1.T, Wv1.T,
                         Wk_lin.T * (1.0 / np.sqrt(DK)), Wk2.T, Wv2, Wv_lin)
    outp, zpart = _scatter_stage(vw, se.reshape(E), dst)
    return _finish_stage(outp, zpart.T[:N])
